# baseline (speedup 1.0000x reference)
import functools

import jax
import jax.numpy as jnp
from jax.experimental import pallas as pl
from jax.experimental.pallas import tpu as pltpu

_B, _T = 32, 32
_Z = 256
_H = 512
_M = 256
_W = 64
_R = 4
_CLIP = 20.0
_CTR = 0.2
_EPS = 1e-6
_IFPAD = 640

_INTERPRET = False
_F32 = jnp.float32
_BF16 = jnp.bfloat16


def _mm_act_kernel(x_ref, w_ref, b_ref, o_ref, *, act):
    y = jnp.dot(x_ref[...], w_ref[...], preferred_element_type=jnp.float32)
    y = y + b_ref[...]
    if act == "lrelu":
        y = jnp.where(y > 0, y, 0.01 * y)
    o_ref[...] = y.astype(o_ref.dtype)


def _mm_act(x, w, b, act, block_rows, out_dtype):
    n, k = x.shape
    _, m = w.shape
    grid = (n // block_rows,)
    return pl.pallas_call(
        functools.partial(_mm_act_kernel, act=act),
        grid=grid,
        in_specs=[
            pl.BlockSpec((block_rows, k), lambda i: (i, 0)),
            pl.BlockSpec((k, m), lambda i: (0, 0)),
            pl.BlockSpec((1, m), lambda i: (0, 0)),
        ],
        out_specs=pl.BlockSpec((block_rows, m), lambda i: (i, 0)),
        out_shape=jax.ShapeDtypeStruct((n, m), out_dtype),
        compiler_params=pltpu.CompilerParams(
            dimension_semantics=("arbitrary",),
        ),
        interpret=_INTERPRET,
    )(x, w, b)


def _mlp_chain_kernel(x_ref, w1_ref, b1_ref, w2_ref, b2_ref, w3_ref, b3_ref,
                      o_ref):
    h = jnp.dot(x_ref[...], w1_ref[...], preferred_element_type=jnp.float32)
    h = (h + b1_ref[...]).astype(_BF16)
    h = jnp.dot(h, w2_ref[...], preferred_element_type=jnp.float32)
    h = (h + b2_ref[...]).astype(_BF16)
    h = jnp.dot(h, w3_ref[...], preferred_element_type=jnp.float32)
    o_ref[...] = jnp.tanh(h + b3_ref[...])


def _mlp_chain(x, w1, b1, w2, b2, w3, b3, block_rows):
    n, k = x.shape
    grid = (n // block_rows,)
    z = w3.shape[1]
    cols = lambda w: w.shape[1]
    return pl.pallas_call(
        _mlp_chain_kernel,
        grid=grid,
        in_specs=[
            pl.BlockSpec((block_rows, k), lambda i: (i, 0)),
            pl.BlockSpec((k, cols(w1)), lambda i: (0, 0)),
            pl.BlockSpec((1, cols(w1)), lambda i: (0, 0)),
            pl.BlockSpec((cols(w1), cols(w2)), lambda i: (0, 0)),
            pl.BlockSpec((1, cols(w2)), lambda i: (0, 0)),
            pl.BlockSpec((cols(w2), z), lambda i: (0, 0)),
            pl.BlockSpec((1, z), lambda i: (0, 0)),
        ],
        out_specs=pl.BlockSpec((block_rows, z), lambda i: (i, 0)),
        out_shape=jax.ShapeDtypeStruct((n, z), _F32),
        compiler_params=pltpu.CompilerParams(
            dimension_semantics=("arbitrary",),
        ),
        interpret=_INTERPRET,
    )(x, w1, b1, w2, b2, w3, b3)


def _im2col_nchw(x):
    n, c, hin, win = x.shape
    ho = hin // 2
    xp = jnp.pad(x, ((0, 0), (0, 0), (1, 1), (1, 1)))
    slabs = [xp[:, :, kh:kh + hin:2, kw:kw + win:2]
             for kh in range(3) for kw in range(3)]
    p = jnp.stack(slabs, axis=0)
    p = p.transpose(1, 3, 4, 2, 0)
    return p.reshape(n * ho * ho, c * 9).astype(_BF16)


def _im2col_nhwc(x):
    n, hin, win, c = x.shape
    ho = hin // 2
    xp = jnp.pad(x, ((0, 0), (1, 1), (1, 1), (0, 0)))
    slabs = [xp[:, kh:kh + hin:2, kw:kw + win:2, :]
             for kh in range(3) for kw in range(3)]
    p = jnp.stack(slabs, axis=0)
    p = p.transpose(1, 2, 3, 4, 0)
    return p.reshape(n * ho * ho, c * 9)


def _encoder(x, p):
    n = x.shape[0]
    w1 = p["c1_w"].reshape(64, 27).T.astype(_BF16)
    y = _mm_act(_im2col_nchw(x), w1, p["c1_b"][None], "lrelu",
                block_rows=16384, out_dtype=_BF16)
    y = y.reshape(n, 16, 16, 64)

    w2 = p["c2_w"].reshape(128, 576).T.astype(_BF16)
    y = _mm_act(_im2col_nhwc(y), w2, p["c2_b"][None], "lrelu",
                block_rows=4096, out_dtype=_BF16)
    y = y.reshape(n, 8, 8, 128)

    w3 = p["c3_w"].reshape(256, 1152).T.astype(_BF16)
    y = _mm_act(_im2col_nhwc(y), w3, p["c3_b"][None], "lrelu",
                block_rows=2048, out_dtype=_BF16)
    y = y.reshape(n, 4096)

    l1 = p["l1_w"].reshape(2048, 256, 4, 4).transpose(0, 2, 3, 1)
    l1 = l1.reshape(2048, 4096).T.astype(_BF16)
    l2 = p["l2_w"].T.astype(_BF16)
    l3 = p["l3_w"].T.astype(_BF16)
    return _mlp_chain(y, l1, p["l1_b"][None], l2, p["l2_b"][None],
                      l3, p["l3_b"][None], block_rows=256)


def _sigmoid(x):
    return jax.nn.sigmoid(x)


def _softplus(x):
    return jnp.log1p(jnp.exp(-jnp.abs(x))) + jnp.maximum(x, 0.0)


def _scan_kernel(enc_ref, h0_ref, a_e_ref, a_r_ref, whh0_ref, wih1_ref,
                 whh1_ref, b0_ref, b1_ref, wif_ref, bif_ref, wyh_ref,
                 wyr_ref, yb_ref, tri_ref, eye_ref,
                 out_ref,
                 memT_s, link_s, prec_s, usage_s, ww_s, rv_s, rw_s, h_s, c_s):
    bb = memT_s.shape[0]

    memT_s[...] = jnp.full(memT_s.shape, _EPS, _F32)
    link_s[...] = jnp.zeros(link_s.shape, _F32)
    prec_s[...] = jnp.zeros(prec_s.shape, _F32)
    usage_s[...] = jnp.zeros(usage_s.shape, _F32)
    ww_s[...] = jnp.zeros(ww_s.shape, _F32)
    rv_s[...] = jnp.zeros(rv_s.shape, _F32)
    rw_s[...] = jnp.zeros(rw_s.shape, _F32)
    h_s[...] = h0_ref[...]
    c_s[...] = h0_ref[...]

    tri = tri_ref[...]
    ney = eye_ref[...]

    def step(t, _):
        enc_t = enc_ref[t]
        rv = rv_s[...]

        g0 = (jnp.dot(enc_t.astype(_BF16), a_e_ref[...],
                      preferred_element_type=_F32)
              + jnp.dot(rv.astype(_BF16), a_r_ref[...],
                        preferred_element_type=_F32)
              + jnp.dot(h_s[0].astype(_BF16), whh0_ref[...],
                        preferred_element_type=_F32)
              + b0_ref[...])
        i0 = _sigmoid(g0[:, 0:_H])
        f0 = _sigmoid(g0[:, _H:2 * _H])
        z0 = jnp.tanh(g0[:, 2 * _H:3 * _H])
        o0 = _sigmoid(g0[:, 3 * _H:4 * _H])
        c0 = f0 * c_s[0] + i0 * z0
        h0n = o0 * jnp.tanh(c0)
        h_s[0] = h0n
        c_s[0] = c0

        g1 = (jnp.dot(h0n.astype(_BF16), wih1_ref[...],
                      preferred_element_type=_F32)
              + jnp.dot(h_s[1].astype(_BF16), whh1_ref[...],
                        preferred_element_type=_F32)
              + b1_ref[...])
        i1 = _sigmoid(g1[:, 0:_H])
        f1 = _sigmoid(g1[:, _H:2 * _H])
        z1 = jnp.tanh(g1[:, 2 * _H:3 * _H])
        o1 = _sigmoid(g1[:, 3 * _H:4 * _H])
        c1 = f1 * c_s[1] + i1 * z1
        h1n = o1 * jnp.tanh(c1)
        h_s[1] = h1n
        c_s[1] = c1

        out = jnp.clip(h1n, -_CLIP, _CLIP)

        v = (jnp.dot(out.astype(_BF16), wif_ref[...],
                     preferred_element_type=_F32) + bif_ref[...])
        read_keys = v[:, 0:256]
        write_key = v[:, 256:320]
        erase = _sigmoid(v[:, 320:384])
        write_vec = v[:, 384:448]
        read_str = _softplus(v[:, 448:452])
        write_str = _softplus(v[:, 452:453])
        free_g = _sigmoid(v[:, 453:457])
        alloc_g = _sigmoid(v[:, 457:458])
        write_g = _sigmoid(v[:, 458:459])

        usage = usage_s[...]
        usage = usage + (1.0 - usage) * ww_s[...]
        rw_old = rw_s[...]
        psi = jnp.ones((bb, _M), _F32)
        for r in range(_R):
            psi = psi * (1.0 - free_g[:, r:r + 1] * rw_old[r])
        usage = usage * psi
        usage_s[...] = usage

        u = _EPS + (1.0 - _EPS) * usage
        logu = jnp.log(u)
        u_i = u[:, :, None]
        u_j = u[:, None, :]
        lu_j = logu[:, None, :]
        lt = u_j < u_i
        eq = u_j == u_i
        contrib = jnp.where(lt, lu_j, 0.0) + jnp.where(eq, lu_j, 0.0) * tri[None]
        s = jnp.sum(contrib, axis=2)
        alloc = (1.0 - u) * jnp.exp(s)

        memT = memT_s[...]
        mem_nrm = jnp.sqrt(jnp.sum(memT * memT, axis=1))
        wk_nrm = jnp.sqrt(jnp.sum(write_key * write_key, axis=1,
                                  keepdims=True))
        wdot = jnp.sum(memT * write_key[:, :, None], axis=1)
        wsim = wdot / (wk_nrm * mem_nrm + _EPS) * write_str
        wsim = wsim - jnp.max(wsim, axis=1, keepdims=True)
        wexp = jnp.exp(wsim)
        wcw = wexp / jnp.sum(wexp, axis=1, keepdims=True)

        ww = write_g * (alloc_g * alloc + (1.0 - alloc_g) * wcw)
        ww_s[...] = ww

        memT = (memT * (1.0 - ww[:, None, :] * erase[:, :, None])
                + ww[:, None, :] * write_vec[:, :, None])
        memT_s[...] = memT

        prec = prec_s[...]
        link = link_s[...]
        wv_i = ww[:, :, None]
        wv_j = ww[:, None, :]
        link = ((1.0 - wv_i - wv_j) * link + wv_i * prec[:, None, :]) * ney[None]
        link_s[...] = link
        prec = (1.0 - jnp.sum(ww, axis=1, keepdims=True)) * prec + ww
        prec_s[...] = prec

        mem_nrm2 = jnp.sqrt(jnp.sum(memT * memT, axis=1))
        rv_parts = []
        for r in range(_R):
            rw_r = rw_old[r]
            fw = jnp.sum(rw_r[:, :, None] * link, axis=1)
            bw = jnp.sum(link * rw_r[:, None, :], axis=2)
            rk = read_keys[:, r * 64:(r + 1) * 64]
            rk_nrm = jnp.sqrt(jnp.sum(rk * rk, axis=1, keepdims=True))
            rdot = jnp.sum(memT * rk[:, :, None], axis=1)
            rsim = rdot / (rk_nrm * mem_nrm2 + _EPS) * read_str[:, r:r + 1]
            rsim = rsim - jnp.max(rsim, axis=1, keepdims=True)
            rex = jnp.exp(rsim)
            rcw = rex / jnp.sum(rex, axis=1, keepdims=True)

            m0 = v[:, 459 + 3 * r:460 + 3 * r]
            m1 = v[:, 460 + 3 * r:461 + 3 * r]
            m2 = v[:, 461 + 3 * r:462 + 3 * r]
            mx = jnp.maximum(jnp.maximum(m0, m1), m2)
            e0 = jnp.exp(m0 - mx)
            e1 = jnp.exp(m1 - mx)
            e2 = jnp.exp(m2 - mx)
            es = e0 + e1 + e2
            rw_new = (e0 * bw + e1 * rcw + e2 * fw) / es
            rw_s[r] = rw_new
            rv_parts.append(jnp.sum(memT * rw_new[:, None, :], axis=2))
        rv_new = jnp.concatenate(rv_parts, axis=1)
        rv_s[...] = rv_new

        y = (jnp.dot(out.astype(_BF16), wyh_ref[...],
                     preferred_element_type=_F32)
             + jnp.dot(rv_new.astype(_BF16), wyr_ref[...],
                       preferred_element_type=_F32)
             + yb_ref[...])
        out_ref[t] = y
        return 0

    jax.lax.fori_loop(0, _T, step, 0)


def _run_scan(enc_tbz, h0, params):
    b = _B
    bb = b // 2

    wih0 = params["Wih0"]
    a_e = wih0[:, 0:256].T.astype(_BF16)
    a_r = wih0[:, 258:514].T.astype(_BF16)
    b0 = (params["bih0"] + params["bhh0"] + wih0[:, 257])[None]
    whh0 = params["Whh0"].T.astype(_BF16)
    wih1 = params["Wih1"].T.astype(_BF16)
    whh1 = params["Whh1"].T.astype(_BF16)
    b1 = (params["bih1"] + params["bhh1"])[None]

    ifw = params["iface_w"]
    ifb = params["iface_b"]
    seg = [(0, 256), (260, 324), (325, 389), (389, 453), (256, 260),
           (324, 325), (453, 457), (457, 458), (458, 459), (459, 471)]
    rows = jnp.concatenate([ifw[a:c] for a, c in seg]
                           + [jnp.zeros((_IFPAD - 471, _H), _F32)], axis=0)
    wif = rows.T.astype(_BF16)
    bif = jnp.concatenate([ifb[a:c] for a, c in seg]
                          + [jnp.zeros((_IFPAD - 471,), _F32)])[None]

    out_w = params["out_w"]
    wyh = (out_w[:, :512] * _CTR).T.astype(_BF16)
    wyr = out_w[:, 512:].T.astype(_BF16)
    yb = params["out_b"][None]

    ii = jax.lax.broadcasted_iota(jnp.int32, (_M, _M), 0)
    jj = jax.lax.broadcasted_iota(jnp.int32, (_M, _M), 1)
    tri = (jj < ii).astype(_F32)
    ney = 1.0 - jnp.eye(_M, dtype=_F32)

    h0s = jnp.swapaxes(h0, 0, 1)
    del h0s

    full = lambda shape: pl.BlockSpec(shape, lambda i: tuple(0 for _ in shape))

    grid = (2,)
    out = pl.pallas_call(
        _scan_kernel,
        grid=grid,
        in_specs=[
            pl.BlockSpec((_T, bb, _Z), lambda i: (0, i, 0)),
            pl.BlockSpec((2, bb, _H), lambda i: (0, i, 0)),
            full((256, 2048)), full((256, 2048)), full((512, 2048)),
            full((512, 2048)), full((512, 2048)),
            full((1, 2048)), full((1, 2048)),
            full((512, _IFPAD)), full((1, _IFPAD)),
            full((512, 256)), full((256, 256)), full((1, 256)),
            full((_M, _M)), full((_M, _M)),
        ],
        out_specs=pl.BlockSpec((_T, bb, _Z), lambda i: (0, i, 0)),
        out_shape=jax.ShapeDtypeStruct((_T, _B, _Z), _F32),
        scratch_shapes=[
            pltpu.VMEM((bb, _W, _M), _F32),
            pltpu.VMEM((bb, _M, _M), _F32),
            pltpu.VMEM((bb, _M), _F32),
            pltpu.VMEM((bb, _M), _F32),
            pltpu.VMEM((bb, _M), _F32),
            pltpu.VMEM((bb, _R * _W), _F32),
            pltpu.VMEM((_R, bb, _M), _F32),
            pltpu.VMEM((2, bb, _H), _F32),
            pltpu.VMEM((2, bb, _H), _F32),
        ],
        compiler_params=pltpu.CompilerParams(
            dimension_semantics=("arbitrary",),
            vmem_limit_bytes=100 * 1024 * 1024,
        ),
        interpret=_INTERPRET,
    )(enc_tbz, h0, a_e, a_r, whh0, wih1, whh1, b0, b1, wif, bif,
      wyh, wyr, yb, tri, ney)
    return out


def kernel(input, h0, params):
    b, t = input.shape[:2]
    enc = _encoder(input.reshape(b * t, *input.shape[2:]), params)
    enc_tbz = jnp.swapaxes(enc.reshape(b, t, _Z), 0, 1)
    ys = _run_scan(enc_tbz, h0, params)
    return jnp.swapaxes(ys, 0, 1)
